# baseline (speedup 1.0000x reference)
import functools

import jax
import jax.numpy as jnp
from jax import lax
from jax.experimental import pallas as pl
from jax.experimental.pallas import tpu as pltpu
from jax.experimental.pallas import tpu_sc as plsc

_LANES = 128
_SUB = 4
_VPAD = 1024
_HALF = _VPAD // 2


def _table_body(embed_ref, w_ref, b_ref, out_ref, wbf_ref, *, vocab):
    rows = embed_ref.shape[0]

    @pl.when(pl.program_id(0) == 0)
    def _():
        wbf_ref[...] = w_ref[...].astype(jnp.bfloat16)

    acc = lax.dot_general(
        embed_ref[...].astype(jnp.bfloat16),
        wbf_ref[...],
        dimension_numbers=(((1,), (1,)), ((), ())),
        preferred_element_type=jnp.float32,
    ) + b_ref[...]
    pad = jnp.zeros((rows, _VPAD - vocab), jnp.float32)
    acc = jnp.concatenate([acc, pad], axis=1)
    lo = lax.bitcast_convert_type(acc[:, :_HALF], jnp.uint32)
    hi = lax.bitcast_convert_type(acc[:, _HALF:], jnp.uint32)
    word = (((hi + 0x8000) & jnp.uint32(0xFFFF0000))
            | ((lo + 0x8000) >> 16))
    pf = lax.bitcast_convert_type(word, jnp.float32)
    for c in range(_SUB):
        out_ref[:, c, :] = pf[:, c * _LANES:(c + 1) * _LANES]


def _make_gather(n_tok, nc, ns):
    nw = nc * ns
    assert n_tok % nw == 0
    b_per_w = n_tok // nw
    chunk = b_per_w // 2
    mesh = plsc.VectorSubcoreMesh(core_axis_name="c", subcore_axis_name="s")

    @functools.partial(
        pl.kernel,
        mesh=mesh,
        compiler_params=pltpu.CompilerParams(use_tc_tiling_on_sc=False),
        out_type=jax.ShapeDtypeStruct((n_tok, _SUB, _LANES), jnp.float32),
        scratch_types=[
            pltpu.VMEM((b_per_w,), jnp.int32),
            pltpu.VMEM((chunk, _SUB, _LANES), jnp.float32),
            pltpu.VMEM((chunk, _SUB, _LANES), jnp.float32),
            pltpu.SemaphoreType.DMA,
            pltpu.SemaphoreType.DMA,
        ],
    )
    def gather(table_hbm, idx_hbm, out_hbm, idx_v, buf0, buf1, sem0, sem1):
        wid = lax.axis_index("s") * nc + lax.axis_index("c")
        base = wid * b_per_w
        pltpu.sync_copy(idx_hbm.at[pl.ds(base, b_per_w)], idx_v)
        d0 = pltpu.async_copy(table_hbm.at[idx_v.at[pl.ds(0, chunk)]],
                              buf0, sem0)
        d1 = pltpu.async_copy(table_hbm.at[idx_v.at[pl.ds(chunk, chunk)]],
                              buf1, sem1)
        d0.wait()
        pltpu.sync_copy(buf0, out_hbm.at[pl.ds(base, chunk)])
        d1.wait()
        pltpu.sync_copy(buf1, out_hbm.at[pl.ds(base + chunk, chunk)])

    return gather


def _transpose_body(g_ref, out_ref, *, vocab):
    for c in range(_SUB):
        xw = lax.bitcast_convert_type(g_ref[:, c, :], jnp.int32)
        xt = xw.T
        lo_f = lax.bitcast_convert_type(xt << 16, jnp.float32)
        hi_f = lax.bitcast_convert_type(xt & jnp.int32(-65536), jnp.float32)
        v0 = c * _LANES
        out_ref[0, v0:v0 + _LANES, :] = lo_f
        v1 = _HALF + c * _LANES
        v1_hi = min(v1 + _LANES, vocab)
        if v1_hi > v1:
            out_ref[0, v1:v1_hi, :] = hi_f[: v1_hi - v1, :]


def kernel(input_ids, embed, W, b):
    batch, seq = input_ids.shape
    vocab, d_model = W.shape
    n_tok = batch * seq

    rows_blk = 40
    n_rblk = vocab // rows_blk
    table3 = pl.pallas_call(
        functools.partial(_table_body, vocab=vocab),
        grid=(n_rblk,),
        in_specs=[
            pl.BlockSpec((rows_blk, d_model), lambda ri: (ri, 0)),
            pl.BlockSpec((vocab, d_model), lambda ri: (0, 0)),
            pl.BlockSpec((1, vocab), lambda ri: (0, 0)),
        ],
        out_specs=pl.BlockSpec((rows_blk, _SUB, _LANES), lambda ri: (ri, 0, 0)),
        out_shape=jax.ShapeDtypeStruct((vocab, _SUB, _LANES), jnp.float32),
        scratch_shapes=[pltpu.VMEM((vocab, d_model), jnp.bfloat16)],
    )(embed, W, b.reshape(1, vocab))

    info = plsc.get_sparse_core_info()
    gather = _make_gather(n_tok, info.num_cores, info.num_subcores)
    ids = input_ids.reshape(n_tok).astype(jnp.int32)
    rows3 = gather(table3, ids)

    s_blk = 2048
    n_sblk = seq // s_blk
    out3 = pl.pallas_call(
        functools.partial(_transpose_body, vocab=vocab),
        grid=(batch, n_sblk),
        in_specs=[
            pl.BlockSpec((s_blk, _SUB, _LANES),
                         lambda bi, si: (bi * n_sblk + si, 0, 0)),
        ],
        out_specs=pl.BlockSpec((1, vocab, s_blk), lambda bi, si: (bi, 0, si)),
        out_shape=jax.ShapeDtypeStruct((batch, vocab, seq), jnp.float32),
    )(rows3)

    return jnp.swapaxes(out3, 1, 2)


# speedup vs baseline: 1.5034x; 1.5034x over previous
import functools

import jax
import jax.numpy as jnp
from jax import lax
from jax.experimental import pallas as pl
from jax.experimental.pallas import tpu as pltpu
from jax.experimental.pallas import tpu_sc as plsc

_LANES = 128
_SUB = 4
_VPAD = 1024
_HALF = _VPAD // 2


def _table_body(embed_ref, w_ref, b_ref, out_ref, wbf_ref, *, vocab):
    rows = embed_ref.shape[0]

    @pl.when(pl.program_id(0) == 0)
    def _():
        wbf_ref[...] = w_ref[...].astype(jnp.bfloat16)

    acc = lax.dot_general(
        embed_ref[...].astype(jnp.bfloat16),
        wbf_ref[...],
        dimension_numbers=(((1,), (1,)), ((), ())),
        preferred_element_type=jnp.float32,
    ) + b_ref[...]
    pad = jnp.zeros((rows, _VPAD - vocab), jnp.float32)
    acc = jnp.concatenate([acc, pad], axis=1)
    lo = lax.bitcast_convert_type(acc[:, :_HALF], jnp.uint32)
    hi = lax.bitcast_convert_type(acc[:, _HALF:], jnp.uint32)
    word = (((hi + 0x8000) & jnp.uint32(0xFFFF0000))
            | ((lo + 0x8000) >> 16))
    pf = lax.bitcast_convert_type(word, jnp.float32)
    for c in range(_SUB):
        out_ref[:, c, :] = pf[:, c * _LANES:(c + 1) * _LANES]


def _make_gather(n_tok, nc, ns):
    nw = nc * ns
    assert n_tok % nw == 0
    b_per_w = n_tok // nw
    chunk = b_per_w // 2
    mesh = plsc.VectorSubcoreMesh(core_axis_name="c", subcore_axis_name="s")

    @functools.partial(
        pl.kernel,
        mesh=mesh,
        compiler_params=pltpu.CompilerParams(use_tc_tiling_on_sc=False),
        out_type=jax.ShapeDtypeStruct((n_tok, _SUB, _LANES), jnp.float32),
        scratch_types=[
            pltpu.VMEM((b_per_w,), jnp.int32),
            pltpu.VMEM((chunk, _SUB, _LANES), jnp.float32),
            pltpu.VMEM((chunk, _SUB, _LANES), jnp.float32),
            pltpu.SemaphoreType.DMA,
            pltpu.SemaphoreType.DMA,
        ],
    )
    def gather(table_hbm, idx_hbm, out_hbm, idx_v, buf0, buf1, sem0, sem1):
        wid = lax.axis_index("s") * nc + lax.axis_index("c")
        base = wid * b_per_w
        pltpu.sync_copy(idx_hbm.at[pl.ds(base, b_per_w)], idx_v)
        d0 = pltpu.async_copy(table_hbm.at[idx_v.at[pl.ds(0, chunk)]],
                              buf0, sem0)
        d1 = pltpu.async_copy(table_hbm.at[idx_v.at[pl.ds(chunk, chunk)]],
                              buf1, sem1)
        d0.wait()
        pltpu.sync_copy(buf0, out_hbm.at[pl.ds(base, chunk)])
        d1.wait()
        pltpu.sync_copy(buf1, out_hbm.at[pl.ds(base + chunk, chunk)])

    return gather


def _transpose_body(g_ref, out_ref, *, vocab):
    for c in range(_SUB):
        xw = lax.bitcast_convert_type(g_ref[:, c, :], jnp.int32)
        xt = xw.T
        lo_f = lax.bitcast_convert_type(xt << 16, jnp.float32)
        hi_f = lax.bitcast_convert_type(xt & jnp.int32(-65536), jnp.float32)
        v0 = c * _LANES
        out_ref[0, v0:v0 + _LANES, :] = lo_f
        v1 = _HALF + c * _LANES
        v1_hi = min(v1 + _LANES, vocab)
        if v1_hi > v1:
            out_ref[0, v1:v1_hi, :] = hi_f[: v1_hi - v1, :]


def kernel(input_ids, embed, W, b):
    batch, seq = input_ids.shape
    vocab, d_model = W.shape
    n_tok = batch * seq

    rows_blk = 200
    n_rblk = vocab // rows_blk
    table3 = pl.pallas_call(
        functools.partial(_table_body, vocab=vocab),
        grid=(n_rblk,),
        in_specs=[
            pl.BlockSpec((rows_blk, d_model), lambda ri: (ri, 0)),
            pl.BlockSpec((vocab, d_model), lambda ri: (0, 0)),
            pl.BlockSpec((1, vocab), lambda ri: (0, 0)),
        ],
        out_specs=pl.BlockSpec((rows_blk, _SUB, _LANES), lambda ri: (ri, 0, 0)),
        out_shape=jax.ShapeDtypeStruct((vocab, _SUB, _LANES), jnp.float32),
        scratch_shapes=[pltpu.VMEM((vocab, d_model), jnp.bfloat16)],
    )(embed, W, b.reshape(1, vocab))

    info = plsc.get_sparse_core_info()
    gather = _make_gather(n_tok, info.num_cores, info.num_subcores)
    ids = input_ids.reshape(n_tok).astype(jnp.int32)
    rows3 = gather(table3, ids)

    s_blk = 2048
    n_sblk = seq // s_blk
    out3 = pl.pallas_call(
        functools.partial(_transpose_body, vocab=vocab),
        grid=(batch, n_sblk),
        in_specs=[
            pl.BlockSpec((s_blk, _SUB, _LANES),
                         lambda bi, si: (bi * n_sblk + si, 0, 0)),
        ],
        out_specs=pl.BlockSpec((1, vocab, s_blk), lambda bi, si: (bi, 0, si)),
        out_shape=jax.ShapeDtypeStruct((batch, vocab, seq), jnp.float32),
    )(rows3)

    return jnp.swapaxes(out3, 1, 2)
